# Initial kernel scaffold; baseline (speedup 1.0000x reference)
import jax
import jax.numpy as jnp
from jax.experimental import pallas as pl


def kernel(x, theta):
    raise NotImplementedError("write your pallas kernel here")


# speedup vs baseline: 2.1487x; 2.1487x over previous
import functools

import jax
import jax.numpy as jnp
from jax import lax
from jax.experimental import pallas as pl
from jax.experimental.pallas import tpu as pltpu
from jax.experimental.pallas import tpu_sc as plsc

N, C, H, W = 8, 96, 224, 224
NUM_TILES = 32
ROWS_PER_TILE = (N * C) // NUM_TILES
TILES_PER_BATCH = NUM_TILES // N
HROWS = 16
NCHUNK = H // HROWS
NLANE = W // 16


def _round_bf16(v):
    b = lax.bitcast_convert_type(v, jnp.int32)
    b = (b + 0x7FFF + ((b >> 16) & 1)) & jnp.int32(-65536)
    return lax.bitcast_convert_type(b, jnp.float32)


def _prep_body(theta_ref, y0_ref, y1_ref, x0_ref, x1_ref,
               w00_ref, w01_ref, w10_ref, w11_ref):
    t00 = theta_ref[0, 0, 0]
    t01 = theta_ref[0, 0, 1]
    t02 = theta_ref[0, 0, 2]
    t10 = theta_ref[0, 1, 0]
    t11 = theta_ref[0, 1, 1]
    t12 = theta_ref[0, 1, 2]
    xi = lax.broadcasted_iota(jnp.int32, (H, W), 1).astype(jnp.float32)
    yi = lax.broadcasted_iota(jnp.int32, (H, W), 0).astype(jnp.float32)
    xs = _round_bf16((xi + 0.5) * (2.0 / W) - 1.0)
    ys = _round_bf16((yi + 0.5) * (2.0 / H) - 1.0)
    tb = [_round_bf16(jnp.full((H, W), t, dtype=jnp.float32))
          for t in (t00, t01, t02, t10, t11, t12)]
    gx = tb[0] * xs + tb[1] * ys + tb[2]
    gy = tb[3] * xs + tb[4] * ys + tb[5]
    ix = ((gx + 1.0) * W - 1.0) * 0.5
    iy = ((gy + 1.0) * H - 1.0) * 0.5
    ix0f = jnp.floor(ix)
    iy0f = jnp.floor(iy)
    wx1 = ix - ix0f
    wx0 = 1.0 - wx1
    wy1 = iy - iy0f
    wy0 = 1.0 - wy1
    ix0 = ix0f.astype(jnp.int32)
    iy0 = iy0f.astype(jnp.int32)
    ix1 = ix0 + 1
    iy1 = iy0 + 1
    fx0 = jnp.where((ix0 >= 0) & (ix0 < W), wx0, 0.0)
    fx1 = jnp.where((ix1 >= 0) & (ix1 < W), wx1, 0.0)
    fy0 = jnp.where((iy0 >= 0) & (iy0 < H), wy0, 0.0)
    fy1 = jnp.where((iy1 >= 0) & (iy1 < H), wy1, 0.0)
    x0_ref[0] = jnp.clip(ix0, 0, W - 1)
    x1_ref[0] = jnp.clip(ix1, 0, W - 1)
    y0_ref[0] = jnp.clip(iy0, 0, H - 1)
    y1_ref[0] = jnp.clip(iy1, 0, H - 1)
    w00_ref[0] = fy0 * fx0
    w01_ref[0] = fy0 * fx1
    w10_ref[0] = fy1 * fx0
    w11_ref[0] = fy1 * fx1


def _prep(theta):
    ishape = jax.ShapeDtypeStruct((N, H, W), jnp.int32)
    fshape = jax.ShapeDtypeStruct((N, H, W), jnp.float32)
    blk = pl.BlockSpec((1, H, W), lambda n: (n, 0, 0))
    return pl.pallas_call(
        _prep_body,
        grid=(N,),
        in_specs=[pl.BlockSpec((1, 2, 3), lambda n: (n, 0, 0),
                               memory_space=pltpu.SMEM)],
        out_specs=[blk] * 8,
        out_shape=[ishape, ishape, ishape, ishape,
                   fshape, fshape, fshape, fshape],
    )(theta)


def _sc_body(x_hbm, y0_hbm, y1_hbm, x0_hbm, x1_hbm,
             w00_hbm, w01_hbm, w10_hbm, w11_hbm, out_hbm,
             img, y0b, y1b, x0b, x1b, w00b, w01b, w10b, w11b, outb,
             sem_img, sem_sb, sem_out):
    wid = lax.axis_index("s") * 2 + lax.axis_index("c")
    n = wid // TILES_PER_BATCH
    c0 = (wid % TILES_PER_BATCH) * ROWS_PER_TILE

    def chan_body(ci, _):
        c = c0 + ci
        cp_img = pltpu.make_async_copy(x_hbm.at[n, c], img, sem_img)
        cp_img.start()
        cp_img.wait()

        def chunk_body(kc, _):
            h0 = kc * HROWS
            sbs = [
                pltpu.make_async_copy(y0_hbm.at[n, pl.ds(h0, HROWS)], y0b, sem_sb),
                pltpu.make_async_copy(y1_hbm.at[n, pl.ds(h0, HROWS)], y1b, sem_sb),
                pltpu.make_async_copy(x0_hbm.at[n, pl.ds(h0, HROWS)], x0b, sem_sb),
                pltpu.make_async_copy(x1_hbm.at[n, pl.ds(h0, HROWS)], x1b, sem_sb),
                pltpu.make_async_copy(w00_hbm.at[n, pl.ds(h0, HROWS)], w00b, sem_sb),
                pltpu.make_async_copy(w01_hbm.at[n, pl.ds(h0, HROWS)], w01b, sem_sb),
                pltpu.make_async_copy(w10_hbm.at[n, pl.ds(h0, HROWS)], w10b, sem_sb),
                pltpu.make_async_copy(w11_hbm.at[n, pl.ds(h0, HROWS)], w11b, sem_sb),
            ]
            for cp in sbs:
                cp.start()
            for cp in sbs:
                cp.wait()

            def row_body(r, _):
                def vec_body(vb, _):
                    s = vb * 16
                    iy0 = y0b[r, pl.ds(s, 16)]
                    iy1 = y1b[r, pl.ds(s, 16)]
                    ix0 = x0b[r, pl.ds(s, 16)]
                    ix1 = x1b[r, pl.ds(s, 16)]
                    w00 = w00b[r, pl.ds(s, 16)]
                    w01 = w01b[r, pl.ds(s, 16)]
                    w10 = w10b[r, pl.ds(s, 16)]
                    w11 = w11b[r, pl.ds(s, 16)]
                    v00 = plsc.load_gather(img, [iy0, ix0])
                    v01 = plsc.load_gather(img, [iy0, ix1])
                    v10 = plsc.load_gather(img, [iy1, ix0])
                    v11 = plsc.load_gather(img, [iy1, ix1])
                    acc = w00 * v00 + w01 * v01 + w10 * v10 + w11 * v11
                    outb[r, pl.ds(s, 16)] = acc
                    return ()

                lax.fori_loop(0, NLANE, vec_body, ())
                return ()

            lax.fori_loop(0, HROWS, row_body, ())
            cp_out = pltpu.make_async_copy(
                outb, out_hbm.at[n, c, pl.ds(h0, HROWS)], sem_out)
            cp_out.start()
            cp_out.wait()
            return ()

        lax.fori_loop(0, NCHUNK, chunk_body, ())
        return ()

    lax.fori_loop(0, ROWS_PER_TILE, chan_body, ())


@functools.cache
def _sc_main():
    ichunk = pltpu.VMEM((HROWS, W), jnp.int32)
    fchunk = pltpu.VMEM((HROWS, W), jnp.float32)
    return pl.kernel(
        _sc_body,
        out_type=jax.ShapeDtypeStruct((N, C, H, W), jnp.float32),
        mesh=plsc.VectorSubcoreMesh(core_axis_name="c", subcore_axis_name="s"),
        compiler_params=pltpu.CompilerParams(use_tc_tiling_on_sc=False,
                                             needs_layout_passes=False),
        scratch_types=[
            pltpu.VMEM((H, W), jnp.float32),
            ichunk, ichunk, ichunk, ichunk,
            fchunk, fchunk, fchunk, fchunk,
            fchunk,
            pltpu.SemaphoreType.DMA,
            pltpu.SemaphoreType.DMA,
            pltpu.SemaphoreType.DMA,
        ],
    )


def kernel(x, theta):
    y0, y1, x0, x1, w00, w01, w10, w11 = _prep(theta)
    return _sc_main()(x, y0, y1, x0, x1, w00, w01, w10, w11)


# speedup vs baseline: 2.7038x; 1.2584x over previous
import functools

import jax
import jax.numpy as jnp
from jax import lax
from jax.experimental import pallas as pl
from jax.experimental.pallas import tpu as pltpu
from jax.experimental.pallas import tpu_sc as plsc

N, C, H, W = 8, 96, 224, 224
NUM_TILES = 32
ROWS_PER_TILE = (N * C) // NUM_TILES
TILES_PER_BATCH = NUM_TILES // N
HROWS = 16
NCHUNK = H // HROWS
NLANE = W // 16


def _round_bf16(v):
    b = lax.bitcast_convert_type(v, jnp.int32)
    b = (b + 0x7FFF + ((b >> 16) & 1)) & jnp.int32(-65536)
    return lax.bitcast_convert_type(b, jnp.float32)


def _prep_body(theta_ref, y0_ref, y1_ref, x0_ref, x1_ref,
               w00_ref, w01_ref, w10_ref, w11_ref):
    t00 = theta_ref[0, 0, 0]
    t01 = theta_ref[0, 0, 1]
    t02 = theta_ref[0, 0, 2]
    t10 = theta_ref[0, 1, 0]
    t11 = theta_ref[0, 1, 1]
    t12 = theta_ref[0, 1, 2]
    xi = lax.broadcasted_iota(jnp.int32, (H, W), 1).astype(jnp.float32)
    yi = lax.broadcasted_iota(jnp.int32, (H, W), 0).astype(jnp.float32)
    xs = _round_bf16((xi + 0.5) * (2.0 / W) - 1.0)
    ys = _round_bf16((yi + 0.5) * (2.0 / H) - 1.0)
    tb = [_round_bf16(jnp.full((H, W), t, dtype=jnp.float32))
          for t in (t00, t01, t02, t10, t11, t12)]
    gx = tb[0] * xs + tb[1] * ys + tb[2]
    gy = tb[3] * xs + tb[4] * ys + tb[5]
    ix = ((gx + 1.0) * W - 1.0) * 0.5
    iy = ((gy + 1.0) * H - 1.0) * 0.5
    ix0f = jnp.floor(ix)
    iy0f = jnp.floor(iy)
    wx1 = ix - ix0f
    wx0 = 1.0 - wx1
    wy1 = iy - iy0f
    wy0 = 1.0 - wy1
    ix0 = ix0f.astype(jnp.int32)
    iy0 = iy0f.astype(jnp.int32)
    ix1 = ix0 + 1
    iy1 = iy0 + 1
    fx0 = jnp.where((ix0 >= 0) & (ix0 < W), wx0, 0.0)
    fx1 = jnp.where((ix1 >= 0) & (ix1 < W), wx1, 0.0)
    fy0 = jnp.where((iy0 >= 0) & (iy0 < H), wy0, 0.0)
    fy1 = jnp.where((iy1 >= 0) & (iy1 < H), wy1, 0.0)
    x0_ref[0] = jnp.clip(ix0, 0, W - 1)
    x1_ref[0] = jnp.clip(ix1, 0, W - 1)
    y0_ref[0] = jnp.clip(iy0, 0, H - 1)
    y1_ref[0] = jnp.clip(iy1, 0, H - 1)
    w00_ref[0] = fy0 * fx0
    w01_ref[0] = fy0 * fx1
    w10_ref[0] = fy1 * fx0
    w11_ref[0] = fy1 * fx1


def _prep(theta):
    ishape = jax.ShapeDtypeStruct((N, H, W), jnp.int32)
    fshape = jax.ShapeDtypeStruct((N, H, W), jnp.float32)
    blk = pl.BlockSpec((1, H, W), lambda n: (n, 0, 0))
    return pl.pallas_call(
        _prep_body,
        grid=(N,),
        in_specs=[pl.BlockSpec((1, 2, 3), lambda n: (n, 0, 0),
                               memory_space=pltpu.SMEM)],
        out_specs=[blk] * 8,
        out_shape=[ishape, ishape, ishape, ishape,
                   fshape, fshape, fshape, fshape],
    )(theta)


def _sc_body(x_hbm, y0_hbm, y1_hbm, x0_hbm, x1_hbm,
             w00_hbm, w01_hbm, w10_hbm, w11_hbm, out_hbm,
             img, y0b, y1b, x0b, x1b, w00b, w01b, w10b, w11b, outb,
             sem_img, sem_sb, sem_out):
    wid = lax.axis_index("s") * 2 + lax.axis_index("c")
    n = wid // TILES_PER_BATCH
    c0 = (wid % TILES_PER_BATCH) * ROWS_PER_TILE

    def chan_body(ci, _):
        c = c0 + ci
        cp_img = pltpu.make_async_copy(x_hbm.at[n, c], img, sem_img)
        cp_img.start()
        cp_img.wait()

        def chunk_body(kc, _):
            h0 = kc * HROWS
            sbs = [
                pltpu.make_async_copy(y0_hbm.at[n, pl.ds(h0, HROWS)], y0b, sem_sb),
                pltpu.make_async_copy(y1_hbm.at[n, pl.ds(h0, HROWS)], y1b, sem_sb),
                pltpu.make_async_copy(x0_hbm.at[n, pl.ds(h0, HROWS)], x0b, sem_sb),
                pltpu.make_async_copy(x1_hbm.at[n, pl.ds(h0, HROWS)], x1b, sem_sb),
                pltpu.make_async_copy(w00_hbm.at[n, pl.ds(h0, HROWS)], w00b, sem_sb),
                pltpu.make_async_copy(w01_hbm.at[n, pl.ds(h0, HROWS)], w01b, sem_sb),
                pltpu.make_async_copy(w10_hbm.at[n, pl.ds(h0, HROWS)], w10b, sem_sb),
                pltpu.make_async_copy(w11_hbm.at[n, pl.ds(h0, HROWS)], w11b, sem_sb),
            ]
            for cp in sbs:
                cp.start()
            for cp in sbs:
                cp.wait()

            def row_body(r, _):
                for vb in range(NLANE):
                    s = vb * 16
                    iy0 = y0b[r, pl.ds(s, 16)]
                    iy1 = y1b[r, pl.ds(s, 16)]
                    ix0 = x0b[r, pl.ds(s, 16)]
                    ix1 = x1b[r, pl.ds(s, 16)]
                    w00 = w00b[r, pl.ds(s, 16)]
                    w01 = w01b[r, pl.ds(s, 16)]
                    w10 = w10b[r, pl.ds(s, 16)]
                    w11 = w11b[r, pl.ds(s, 16)]
                    v00 = plsc.load_gather(img, [iy0, ix0])
                    v01 = plsc.load_gather(img, [iy0, ix1])
                    v10 = plsc.load_gather(img, [iy1, ix0])
                    v11 = plsc.load_gather(img, [iy1, ix1])
                    acc = w00 * v00 + w01 * v01 + w10 * v10 + w11 * v11
                    outb[r, pl.ds(s, 16)] = acc
                return ()

            lax.fori_loop(0, HROWS, row_body, ())
            cp_out = pltpu.make_async_copy(
                outb, out_hbm.at[n, c, pl.ds(h0, HROWS)], sem_out)
            cp_out.start()
            cp_out.wait()
            return ()

        lax.fori_loop(0, NCHUNK, chunk_body, ())
        return ()

    lax.fori_loop(0, ROWS_PER_TILE, chan_body, ())


@functools.cache
def _sc_main():
    ichunk = pltpu.VMEM((HROWS, W), jnp.int32)
    fchunk = pltpu.VMEM((HROWS, W), jnp.float32)
    return pl.kernel(
        _sc_body,
        out_type=jax.ShapeDtypeStruct((N, C, H, W), jnp.float32),
        mesh=plsc.VectorSubcoreMesh(core_axis_name="c", subcore_axis_name="s"),
        compiler_params=pltpu.CompilerParams(use_tc_tiling_on_sc=False,
                                             needs_layout_passes=False),
        scratch_types=[
            pltpu.VMEM((H, W), jnp.float32),
            ichunk, ichunk, ichunk, ichunk,
            fchunk, fchunk, fchunk, fchunk,
            fchunk,
            pltpu.SemaphoreType.DMA,
            pltpu.SemaphoreType.DMA,
            pltpu.SemaphoreType.DMA,
        ],
    )


def kernel(x, theta):
    y0, y1, x0, x1, w00, w01, w10, w11 = _prep(theta)
    return _sc_main()(x, y0, y1, x0, x1, w00, w01, w10, w11)
